# Initial kernel scaffold; baseline (speedup 1.0000x reference)
import jax
import jax.numpy as jnp
from jax.experimental import pallas as pl


def kernel(data, labels, x, steps, bc_left, bc_right, c):
    raise NotImplementedError("write your pallas kernel here")


# speedup vs baseline: 15.0225x; 15.0225x over previous
import jax
import jax.numpy as jnp
from jax.experimental import pallas as pl

_NX = 4096
_K = 16
_W = 32
_CH = 512
_TWIN = 5
_NB = 16
_PADX = 2.0e9
_PADI = 1 << 20
_INFD = 3.0e9
_BIGI = 1 << 22


def _graph_body(xr_ref, xc_ref, src_ref):
    xr = xr_ref[...]
    xc = xc_ref[...]
    nchunks = _NX // _CH

    rank_row = jnp.zeros((1, _NX), jnp.int32)
    for c in range(nchunks):
        j0 = c * _CH
        xj = xc[j0:j0 + _CH, :]
        jio = jax.lax.broadcasted_iota(jnp.int32, (_CH, _NX), 0) + j0
        iio = jax.lax.broadcasted_iota(jnp.int32, (_CH, _NX), 1)
        less = (xj < xr) | ((xj == xr) & (jio < iio))
        rank_row = rank_row + jnp.sum(less.astype(jnp.int32), axis=0,
                                      keepdims=True)

    rank_col = jnp.zeros((_NX, 1), jnp.int32)
    for c in range(nchunks):
        j0 = c * _CH
        xj = xr[:, j0:j0 + _CH]
        jio = jax.lax.broadcasted_iota(jnp.int32, (_NX, _CH), 1) + j0
        iio = jax.lax.broadcasted_iota(jnp.int32, (_NX, _CH), 0)
        less = (xj < xc) | ((xj == xc) & (jio < iio))
        rank_col = rank_col + jnp.sum(less.astype(jnp.int32), axis=1,
                                      keepdims=True)

    perm_row = jnp.zeros((1, _NX), jnp.int32)
    sx_row = jnp.zeros((1, _NX), jnp.float32)
    for c in range(nchunks):
        i0 = c * _CH
        rc = rank_col[i0:i0 + _CH, :]
        xi = xc[i0:i0 + _CH, :]
        rio = jax.lax.broadcasted_iota(jnp.int32, (_CH, _NX), 1)
        iio = jax.lax.broadcasted_iota(jnp.int32, (_CH, _NX), 0) + i0
        hit = rc == rio
        perm_row = perm_row + jnp.sum(jnp.where(hit, iio, 0), axis=0,
                                      keepdims=True)
        sx_row = sx_row + jnp.sum(jnp.where(hit, xi, 0.0), axis=0,
                                  keepdims=True)

    padx = jnp.full((1, _W), _PADX, jnp.float32)
    padi = jnp.full((1, _W), _PADI, jnp.int32)
    s_pad = jnp.concatenate([padx, sx_row, padx], axis=1)
    p_pad = jnp.concatenate([padi, perm_row, padi], axis=1)
    cd_rows = []
    ci_rows = []
    for o in range(2 * _W + 1):
        if o == _W:
            continue
        cd_rows.append(jnp.abs(s_pad[:, o:o + _NX] - sx_row))
        ci_rows.append(p_pad[:, o:o + _NX])
    cd = jnp.concatenate(cd_rows, axis=0)
    ci = jnp.concatenate(ci_rows, axis=0)

    nbr_rows = []
    for _ in range(_K):
        dmin = jnp.min(cd, axis=0, keepdims=True)
        sel = jnp.min(jnp.where(cd == dmin, ci, _BIGI), axis=0,
                      keepdims=True)
        nbr_rows.append(sel)
        cd = jnp.where(ci == sel, _INFD, cd)

    nbr = jnp.concatenate(nbr_rows, axis=0).astype(jnp.float32)

    acc = jnp.zeros((_K, _NX), jnp.float32)
    for c in range(nchunks):
        r0 = c * _CH
        rio = jax.lax.broadcasted_iota(jnp.int32, (_CH, _NX), 0) + r0
        onehot = (rio == rank_row).astype(jnp.float32)
        acc = acc + jnp.dot(nbr[:, r0:r0 + _CH], onehot,
                            preferred_element_type=jnp.float32)
    src_ref[...] = acc.astype(jnp.int32)


def _features_body(data_ref, labels_ref, xc_ref, tv_ref, bl_ref, br_ref,
                   cv_ref, u_ref, y_ref, pos_ref, batch_ref, bcl_ref,
                   bcr_ref, cn_ref):
    b = pl.program_id(0)
    u_ref[0] = data_ref[0].T
    y_ref[0] = labels_ref[0].T
    tcol = jnp.full((_NX, 1), tv_ref[0, 0, 0], jnp.float32)
    pos_ref[0] = jnp.concatenate([tcol, xc_ref[...]], axis=1)
    batch_ref[...] = jnp.full((1, 1, _NX), b, jnp.int32)
    bcl_ref[...] = jnp.full((1, 1, _NX), bl_ref[0, 0, 0], jnp.float32)
    bcr_ref[...] = jnp.full((1, 1, _NX), br_ref[0, 0, 0], jnp.float32)
    cn_ref[...] = jnp.full((1, 1, _NX), cv_ref[0, 0, 0], jnp.float32)


def kernel(data, labels, x, steps, bc_left, bc_right, c):
    B, tw, nx = data.shape
    x0_row = x[0].reshape(1, nx)
    x0_col = x[0].reshape(nx, 1)

    src_ti = pl.pallas_call(
        _graph_body,
        out_shape=jax.ShapeDtypeStruct((_K, nx), jnp.int32),
    )(x0_row, x0_col)

    tvals = jnp.linspace(0.0, 1.0, 250, dtype=jnp.float32)[steps]
    scal2 = lambda: pl.BlockSpec((1, 1, 1), lambda b: (b, 0, 0))
    u3, y3, pos3, batch2, bcl2, bcr2, cn2 = pl.pallas_call(
        _features_body,
        grid=(B,),
        in_specs=[
            pl.BlockSpec((1, tw, nx), lambda b: (b, 0, 0)),
            pl.BlockSpec((1, tw, nx), lambda b: (b, 0, 0)),
            pl.BlockSpec((nx, 1), lambda b: (0, 0)),
            scal2(), scal2(), scal2(), scal2(),
        ],
        out_specs=[
            pl.BlockSpec((1, nx, tw), lambda b: (b, 0, 0)),
            pl.BlockSpec((1, nx, tw), lambda b: (b, 0, 0)),
            pl.BlockSpec((1, nx, 2), lambda b: (b, 0, 0)),
            pl.BlockSpec((1, 1, nx), lambda b: (b, 0, 0)),
            pl.BlockSpec((1, 1, nx), lambda b: (b, 0, 0)),
            pl.BlockSpec((1, 1, nx), lambda b: (b, 0, 0)),
            pl.BlockSpec((1, 1, nx), lambda b: (b, 0, 0)),
        ],
        out_shape=[
            jax.ShapeDtypeStruct((B, nx, tw), jnp.float32),
            jax.ShapeDtypeStruct((B, nx, tw), jnp.float32),
            jax.ShapeDtypeStruct((B, nx, 2), jnp.float32),
            jax.ShapeDtypeStruct((B, 1, nx), jnp.int32),
            jax.ShapeDtypeStruct((B, 1, nx), jnp.float32),
            jax.ShapeDtypeStruct((B, 1, nx), jnp.float32),
            jax.ShapeDtypeStruct((B, 1, nx), jnp.float32),
        ],
    )(data, labels, x0_col, tvals.reshape(B, 1, 1), bc_left.reshape(B, 1, 1),
      bc_right.reshape(B, 1, 1), c.reshape(B, 1, 1))

    src0 = src_ti.T.reshape(-1)
    dst0 = jnp.repeat(jnp.arange(nx, dtype=jnp.int32), _K)
    offs = (jnp.arange(B, dtype=jnp.int32) * nx)[:, None]
    src = (src0[None, :] + offs).reshape(-1)
    dst = (dst0[None, :] + offs).reshape(-1)
    edge_index = jnp.stack([src, dst], 0)

    u = u3.reshape(B * nx, tw)
    y = y3.reshape(B * nx, tw)
    pos = pos3.reshape(B * nx, 2)
    batch = batch2.reshape(-1)
    bc_l = bcl2.reshape(B * nx, 1)
    bc_r = bcr2.reshape(B * nx, 1)
    c_n = cn2.reshape(B * nx, 1)
    return (u, edge_index, y, pos, batch, bc_l, bc_r, c_n)
